# baseline (speedup 1.0000x reference)
import functools

import jax
import jax.numpy as jnp
from jax.experimental import pallas as pl

_DEPTH = 10
_DIM = 1024
_NB_SPLIT = 2**_DEPTH - 1
_NB_NODES = 2**(_DEPTH + 1) - 1


def _rep2(v, idx):
    L = v.shape[1]
    if L > 128:
        return jnp.concatenate(
            [_rep2(v[:, c:c + 128], idx) for c in range(0, L, 128)], axis=1)
    return jnp.take_along_axis(v, idx[:, :2 * L], axis=1)


def _tree_body(x_ref, at_ref, o_ref, *, tb):
    xa = jnp.dot(x_ref[:], at_ref[:], preferred_element_type=jnp.float32)

    lane = jax.lax.broadcasted_iota(jnp.int32, (tb, 256), 1)
    idx = lane // 2
    sgn = jnp.where(lane % 2 == 0, 1.0, -1.0)

    o_ref[:, 0:1] = jnp.ones((tb, 1), jnp.float32)
    lvl = jnp.ones((tb, 1), jnp.float32)
    for d in range(_DEPTH):
        L = 1 << d
        xa_d = xa[:, L - 1:2 * L - 1]
        rep_parent = _rep2(lvl, idx)
        rep_score = _rep2(xa_d, idx)
        if 2 * L <= 256:
            rep_score = rep_score * sgn[:, :2 * L]
        else:
            rep_score = rep_score * jnp.concatenate([sgn] * (L // 128), axis=1)
        lvl = jnp.minimum(rep_parent, rep_score)
        o_ref[:, 2 * L - 1:4 * L - 1] = jnp.maximum(lvl, 0.0)


@jax.jit
def kernel(x, A):
    b, dim = x.shape
    a_t = jnp.concatenate(
        [A.T, jnp.zeros((dim, _DIM - _NB_SPLIT), A.dtype)], axis=1)
    tb = 1024
    return pl.pallas_call(
        functools.partial(_tree_body, tb=tb),
        grid=(b // tb,),
        in_specs=[
            pl.BlockSpec((tb, dim), lambda i: (i, 0)),
            pl.BlockSpec((dim, _DIM), lambda i: (0, 0)),
        ],
        out_specs=pl.BlockSpec((tb, _NB_NODES), lambda i: (i, 0)),
        out_shape=jax.ShapeDtypeStruct((b, _NB_NODES), jnp.float32),
    )(x, a_t)


# speedup vs baseline: 2.1113x; 2.1113x over previous
import functools

import jax
import jax.numpy as jnp
from jax.experimental import pallas as pl

_DEPTH = 10
_DIM = 1024
_NB_SPLIT = 2**_DEPTH - 1
_NB_NODES = 2**(_DEPTH + 1) - 1


def _rep2(v, idx):
    L = v.shape[1]
    if L > 128:
        return jnp.concatenate(
            [_rep2(v[:, c:c + 128], idx) for c in range(0, L, 128)], axis=1)
    return jnp.take_along_axis(v, idx[:, :2 * L], axis=1)


def _tree_body(x_ref, at_ref, o_ref, *, tb):
    xa = jnp.dot(x_ref[:], at_ref[:], preferred_element_type=jnp.float32)

    o_ref[:, 0:1] = jnp.ones((tb, 1), jnp.float32)
    lvl = jnp.ones((tb, 1), jnp.float32)
    for d in range(_DEPTH):
        L = 1 << d
        xa_d = xa[:, L - 1:2 * L - 1]
        rows = jax.lax.broadcasted_iota(jnp.int32, (L, 2 * L), 0)
        cols = jax.lax.broadcasted_iota(jnp.int32, (L, 2 * L), 1)
        hit = cols // 2 == rows
        r = jnp.where(hit, 1.0, 0.0)
        rs = jnp.where(hit, jnp.where(cols % 2 == 0, 1.0, -1.0), 0.0)
        rep_parent = jnp.dot(lvl, r, preferred_element_type=jnp.float32)
        rep_score = jnp.dot(xa_d, rs, preferred_element_type=jnp.float32)
        lvl = jnp.minimum(rep_parent, rep_score)
        o_ref[:, 2 * L - 1:4 * L - 1] = jnp.maximum(lvl, 0.0)


@jax.jit
def kernel(x, A):
    b, dim = x.shape
    a_t = jnp.concatenate(
        [A.T, jnp.zeros((dim, _DIM - _NB_SPLIT), A.dtype)], axis=1)
    tb = 1024
    return pl.pallas_call(
        functools.partial(_tree_body, tb=tb),
        grid=(b // tb,),
        in_specs=[
            pl.BlockSpec((tb, dim), lambda i: (i, 0)),
            pl.BlockSpec((dim, _DIM), lambda i: (0, 0)),
        ],
        out_specs=pl.BlockSpec((tb, _NB_NODES), lambda i: (i, 0)),
        out_shape=jax.ShapeDtypeStruct((b, _NB_NODES), jnp.float32),
    )(x, a_t)
